# baseline (speedup 1.0000x reference)
import functools

import jax
import jax.numpy as jnp
from jax import lax
from jax.experimental import pallas as pl
from jax.experimental.pallas import tpu as pltpu
from jax.experimental.pallas import tpu_sc as plsc

BATCH = 16384
FACTOR_NUM = 16
NUM_CORES = 2
NUM_SUBCORES = 16
NUM_WORKERS = NUM_CORES * NUM_SUBCORES
B_PER_W = BATCH // NUM_WORKERS
LANES = 16
NUM_BLOCKS = B_PER_W // LANES


def _fcf_body(user_hbm, item_hbm, users_W_hbm, items_W_hbm, out_hbm,
              uidx_v, iidx_v, urows_v, irows_v, out_v, sem_u, sem_i):
    wid = lax.axis_index("s") * NUM_CORES + lax.axis_index("c")
    base = wid * B_PER_W

    pltpu.sync_copy(user_hbm.at[pl.ds(base, B_PER_W)], uidx_v)
    pltpu.sync_copy(item_hbm.at[pl.ds(base, B_PER_W)], iidx_v)

    cp_u = pltpu.async_copy(users_W_hbm.at[uidx_v], urows_v, sem_u)
    cp_i = pltpu.async_copy(items_W_hbm.at[iidx_v], irows_v, sem_i)
    cp_u.wait()
    cp_i.wait()

    lane = lax.iota(jnp.int32, LANES)

    def block(blk, _):
        row = blk * LANES + lane
        acc = jnp.zeros((LANES,), jnp.float32)
        for d in range(FACTOR_NUM):
            col = jnp.full((LANES,), d, jnp.int32)
            u = plsc.load_gather(urows_v, [row, col])
            v = plsc.load_gather(irows_v, [row, col])
            acc = acc + u * v
        neg = acc < 0.0
        z = jnp.exp(jnp.where(neg, acc, -acc))
        out_v[pl.ds(blk * LANES, LANES)] = jnp.where(neg, z / (1.0 + z),
                                                     1.0 / (1.0 + z))
        return 0

    lax.fori_loop(0, NUM_BLOCKS, block, 0)

    pltpu.sync_copy(out_v, out_hbm.at[pl.ds(base, B_PER_W)])


@jax.jit
def _fcf(user, item, users_W, items_W):
    mesh = plsc.VectorSubcoreMesh(core_axis_name="c", subcore_axis_name="s")
    kern = pl.kernel(
        _fcf_body,
        out_type=jax.ShapeDtypeStruct((BATCH,), jnp.float32),
        mesh=mesh,
        compiler_params=pltpu.CompilerParams(needs_layout_passes=False,
                                             use_tc_tiling_on_sc=False),
        scratch_types=[
            pltpu.VMEM((B_PER_W,), jnp.int32),
            pltpu.VMEM((B_PER_W,), jnp.int32),
            pltpu.VMEM((B_PER_W, FACTOR_NUM), jnp.float32),
            pltpu.VMEM((B_PER_W, FACTOR_NUM), jnp.float32),
            pltpu.VMEM((B_PER_W,), jnp.float32),
            pltpu.SemaphoreType.DMA,
            pltpu.SemaphoreType.DMA,
        ],
    )
    return kern(user, item, users_W, items_W)


def kernel(user, item, users_W, items_W):
    return _fcf(user.astype(jnp.int32), item.astype(jnp.int32),
                users_W, items_W)


# speedup vs baseline: 5.9755x; 5.9755x over previous
import jax
import jax.numpy as jnp
from jax import lax
from jax.experimental import pallas as pl
from jax.experimental.pallas import tpu as pltpu
from jax.experimental.pallas import tpu_sc as plsc

BATCH = 16384
FACTOR_NUM = 16
NUM_CORES = 2
NUM_SUBCORES = 16
NUM_WORKERS = NUM_CORES * NUM_SUBCORES
B_PER_W = BATCH // NUM_WORKERS
LANES = 16
CHUNK = 8
NUM_PAIRS = B_PER_W // (2 * CHUNK)
WIN = 128
IDX_PAD = B_PER_W + LANES


def _fcf_body(user_hbm, item_hbm, users_T_hbm, items_T_hbm, out_hbm,
              uidx_v, iidx_v, uwin_v, iwin_v,
              ustg_v, istg_v, out_v, sem_a, sem_b):
    wid = lax.axis_index("s") * NUM_CORES + lax.axis_index("c")
    base = wid * B_PER_W

    pltpu.sync_copy(user_hbm.at[pl.ds(base, B_PER_W)],
                    uidx_v.at[pl.ds(0, B_PER_W)])
    pltpu.sync_copy(item_hbm.at[pl.ds(base, B_PER_W)],
                    iidx_v.at[pl.ds(0, B_PER_W)])

    lane = lax.iota(jnp.int32, LANES)
    sems = (sem_a, sem_b)

    def fire_chunk(j0, buf):
        uvec = uidx_v[pl.ds(j0, LANES)]
        ivec = iidx_v[pl.ds(j0, LANES)]
        for k in range(CHUNK):
            qu = pl.multiple_of((uvec[k] // WIN) * WIN, WIN)
            qi = pl.multiple_of((ivec[k] // WIN) * WIN, WIN)
            pltpu.make_async_copy(
                users_T_hbm.at[:, pl.ds(qu, WIN)], uwin_v.at[buf, k],
                sems[buf]).start()
            pltpu.make_async_copy(
                items_T_hbm.at[:, pl.ds(qi, WIN)], iwin_v.at[buf, k],
                sems[buf]).start()

    def drain_extract(j0, buf, stg0):
        for k in range(CHUNK):
            pltpu.make_async_copy(
                users_T_hbm.at[:, pl.ds(0, WIN)], uwin_v.at[buf, k],
                sems[buf]).wait()
            pltpu.make_async_copy(
                items_T_hbm.at[:, pl.ds(0, WIN)], iwin_v.at[buf, k],
                sems[buf]).wait()
        uvec = uidx_v[pl.ds(j0, LANES)]
        ivec = iidx_v[pl.ds(j0, LANES)]
        for k in range(CHUNK):
            mu = jnp.full((LANES,), uvec[k] % WIN, jnp.int32)
            mi = jnp.full((LANES,), ivec[k] % WIN, jnp.int32)
            u = plsc.load_gather(uwin_v.at[buf, k], [lane, mu])
            v = plsc.load_gather(iwin_v.at[buf, k], [lane, mi])
            ustg_v[pl.ds((stg0 + k) * FACTOR_NUM, FACTOR_NUM)] = u
            istg_v[pl.ds((stg0 + k) * FACTOR_NUM, FACTOR_NUM)] = v

    fire_chunk(0, 0)
    fire_chunk(CHUNK, 1)

    def step(t, _):
        j0 = t * 2 * CHUNK
        drain_extract(j0, 0, 0)

        @pl.when(j0 + 2 * CHUNK < B_PER_W)
        def _():
            fire_chunk(j0 + 2 * CHUNK, 0)

        drain_extract(j0 + CHUNK, 1, CHUNK)

        @pl.when(j0 + 3 * CHUNK < B_PER_W)
        def _():
            fire_chunk(j0 + 3 * CHUNK, 1)

        acc = jnp.zeros((LANES,), jnp.float32)
        for d in range(FACTOR_NUM):
            off = lane * FACTOR_NUM + d
            acc = acc + plsc.load_gather(ustg_v, [off]) * plsc.load_gather(
                istg_v, [off])
        neg = acc < 0.0
        z = jnp.exp(jnp.where(neg, acc, -acc))
        out_v[pl.ds(j0, LANES)] = jnp.where(neg, z / (1.0 + z), 1.0 / (1.0 + z))
        return 0

    lax.fori_loop(0, NUM_PAIRS, step, 0)

    pltpu.sync_copy(out_v, out_hbm.at[pl.ds(base, B_PER_W)])


@jax.jit
def _fcf(user, item, users_W, items_W):
    mesh = plsc.VectorSubcoreMesh(core_axis_name="c", subcore_axis_name="s")
    kern = pl.kernel(
        _fcf_body,
        out_type=jax.ShapeDtypeStruct((BATCH,), jnp.float32),
        mesh=mesh,
        compiler_params=pltpu.CompilerParams(needs_layout_passes=False,
                                             use_tc_tiling_on_sc=True),
        scratch_types=[
            pltpu.VMEM((IDX_PAD,), jnp.int32),
            pltpu.VMEM((IDX_PAD,), jnp.int32),
            pltpu.VMEM((2, CHUNK, FACTOR_NUM, WIN), jnp.float32),
            pltpu.VMEM((2, CHUNK, FACTOR_NUM, WIN), jnp.float32),
            pltpu.VMEM((2 * CHUNK * FACTOR_NUM,), jnp.float32),
            pltpu.VMEM((2 * CHUNK * FACTOR_NUM,), jnp.float32),
            pltpu.VMEM((B_PER_W,), jnp.float32),
            pltpu.SemaphoreType.DMA,
            pltpu.SemaphoreType.DMA,
        ],
    )
    return kern(user, item, users_W.T, items_W.T)


def kernel(user, item, users_W, items_W):
    return _fcf(user.astype(jnp.int32), item.astype(jnp.int32),
                users_W, items_W)


# speedup vs baseline: 5.9815x; 1.0010x over previous
import jax
import jax.numpy as jnp
from jax import lax
from jax.experimental import pallas as pl
from jax.experimental.pallas import tpu as pltpu
from jax.experimental.pallas import tpu_sc as plsc

BATCH = 16384
FACTOR_NUM = 16
NUM_CORES = 2
NUM_SUBCORES = 16
NUM_WORKERS = NUM_CORES * NUM_SUBCORES
B_PER_W = BATCH // NUM_WORKERS
LANES = 16
CHUNK = 8
NUM_PAIRS = B_PER_W // (2 * CHUNK)
WIN = 128
STG_STRIDE = 17
IDX_PAD = B_PER_W + LANES


def _fcf_body(user_hbm, item_hbm, users_T_hbm, items_T_hbm, out_hbm,
              uidx_v, iidx_v, uwin_v, iwin_v,
              ustg_v, istg_v, out_v, sem_a, sem_b):
    wid = lax.axis_index("s") * NUM_CORES + lax.axis_index("c")
    base = wid * B_PER_W

    pltpu.sync_copy(user_hbm.at[pl.ds(base, B_PER_W)],
                    uidx_v.at[pl.ds(0, B_PER_W)])
    pltpu.sync_copy(item_hbm.at[pl.ds(base, B_PER_W)],
                    iidx_v.at[pl.ds(0, B_PER_W)])

    lane = lax.iota(jnp.int32, LANES)
    sems = (sem_a, sem_b)

    def fire_chunk(j0, buf):
        uvec = uidx_v[pl.ds(j0, LANES)]
        ivec = iidx_v[pl.ds(j0, LANES)]
        for k in range(CHUNK):
            qu = pl.multiple_of((uvec[k] // WIN) * WIN, WIN)
            qi = pl.multiple_of((ivec[k] // WIN) * WIN, WIN)
            pltpu.make_async_copy(
                users_T_hbm.at[:, pl.ds(qu, WIN)], uwin_v.at[buf, k],
                sems[buf]).start()
            pltpu.make_async_copy(
                items_T_hbm.at[:, pl.ds(qi, WIN)], iwin_v.at[buf, k],
                sems[buf]).start()

    def drain_extract(j0, buf, stg0):
        for k in range(CHUNK):
            pltpu.make_async_copy(
                users_T_hbm.at[:, pl.ds(0, WIN)], uwin_v.at[buf, k],
                sems[buf]).wait()
            pltpu.make_async_copy(
                items_T_hbm.at[:, pl.ds(0, WIN)], iwin_v.at[buf, k],
                sems[buf]).wait()
        uvec = uidx_v[pl.ds(j0, LANES)]
        ivec = iidx_v[pl.ds(j0, LANES)]
        for k in range(CHUNK):
            mu = jnp.full((LANES,), uvec[k] % WIN, jnp.int32)
            mi = jnp.full((LANES,), ivec[k] % WIN, jnp.int32)
            u = plsc.load_gather(uwin_v.at[buf, k], [lane, mu])
            v = plsc.load_gather(iwin_v.at[buf, k], [lane, mi])
            ustg_v[pl.ds((stg0 + k) * STG_STRIDE, FACTOR_NUM)] = u
            istg_v[pl.ds((stg0 + k) * STG_STRIDE, FACTOR_NUM)] = v

    fire_chunk(0, 0)
    fire_chunk(CHUNK, 1)

    def step(t, _):
        j0 = t * 2 * CHUNK
        drain_extract(j0, 0, 0)

        @pl.when(j0 + 2 * CHUNK < B_PER_W)
        def _():
            fire_chunk(j0 + 2 * CHUNK, 0)

        drain_extract(j0 + CHUNK, 1, CHUNK)

        @pl.when(j0 + 3 * CHUNK < B_PER_W)
        def _():
            fire_chunk(j0 + 3 * CHUNK, 1)

        acc = jnp.zeros((LANES,), jnp.float32)
        for d in range(FACTOR_NUM):
            off = lane * STG_STRIDE + d
            acc = acc + plsc.load_gather(ustg_v, [off]) * plsc.load_gather(
                istg_v, [off])
        neg = acc < 0.0
        z = jnp.exp(jnp.where(neg, acc, -acc))
        out_v[pl.ds(j0, LANES)] = jnp.where(neg, z / (1.0 + z), 1.0 / (1.0 + z))
        return 0

    lax.fori_loop(0, NUM_PAIRS, step, 0)

    pltpu.sync_copy(out_v, out_hbm.at[pl.ds(base, B_PER_W)])


@jax.jit
def _fcf(user, item, users_W, items_W):
    mesh = plsc.VectorSubcoreMesh(core_axis_name="c", subcore_axis_name="s")
    kern = pl.kernel(
        _fcf_body,
        out_type=jax.ShapeDtypeStruct((BATCH,), jnp.float32),
        mesh=mesh,
        compiler_params=pltpu.CompilerParams(needs_layout_passes=False,
                                             use_tc_tiling_on_sc=True),
        scratch_types=[
            pltpu.VMEM((IDX_PAD,), jnp.int32),
            pltpu.VMEM((IDX_PAD,), jnp.int32),
            pltpu.VMEM((2, CHUNK, FACTOR_NUM, WIN), jnp.float32),
            pltpu.VMEM((2, CHUNK, FACTOR_NUM, WIN), jnp.float32),
            pltpu.VMEM((2 * CHUNK * STG_STRIDE,), jnp.float32),
            pltpu.VMEM((2 * CHUNK * STG_STRIDE,), jnp.float32),
            pltpu.VMEM((B_PER_W,), jnp.float32),
            pltpu.SemaphoreType.DMA,
            pltpu.SemaphoreType.DMA,
        ],
    )
    return kern(user, item, users_W.T, items_W.T)


def kernel(user, item, users_W, items_W):
    return _fcf(user.astype(jnp.int32), item.astype(jnp.int32),
                users_W, items_W)
